# Initial kernel scaffold; baseline (speedup 1.0000x reference)
import jax
import jax.numpy as jnp
from jax.experimental import pallas as pl


def kernel(predicted_trajectories_global, terrain_points):
    raise NotImplementedError("write your pallas kernel here")


# speedup vs baseline: 1.0052x; 1.0052x over previous
import jax
import jax.numpy as jnp
from jax.experimental import pallas as pl
from jax.experimental.pallas import tpu as pltpu

RQ = 2.0
THRESHOLD = 4.0
RADIUS = 2.0 * RQ

TBLK = 2048
NQ = 720


def _body(q_ref, tT_ref, k2_ref, cost_ref, dsum_ref, cnt_ref):
    i = pl.program_id(0)
    nsteps = pl.num_programs(0)

    @pl.when(i == 0)
    def _init():
        dsum_ref[...] = jnp.zeros_like(dsum_ref)
        cnt_ref[...] = jnp.zeros_like(cnt_ref)

    q = q_ref[...]
    g = jax.lax.dot_general(
        q, tT_ref[...], (((1,), (0,)), ((), ())),
        preferred_element_type=jnp.float32)
    q2 = jnp.sum(q * q, axis=1, keepdims=True)
    d2 = jnp.maximum(q2 + k2_ref[...] - 2.0 * g, 0.0)
    dist = jnp.sqrt(d2 + 1e-12)
    m = dist <= RADIUS
    dsum_ref[...] += jnp.sum(jnp.where(m, dist, 0.0), axis=1, keepdims=True)
    cnt_ref[...] += jnp.sum(m.astype(jnp.float32), axis=1, keepdims=True)

    @pl.when(i == nsteps - 1)
    def _fini():
        cnt = cnt_ref[...]
        d_mean = dsum_ref[...] / jnp.maximum(cnt, 1.0)
        cost = -(d_mean * d_mean) * (1.0 / (RQ * RQ)) + THRESHOLD
        cost_ref[...] = jnp.where(cnt > 0.0, cost, 0.0)


def kernel(predicted_trajectories_global, terrain_points):
    traj = predicted_trajectories_global
    B, P, T, D = traj.shape
    q = traj.reshape(-1, D)

    n = terrain_points.shape[0]
    npad = ((n + TBLK - 1) // TBLK) * TBLK
    t = jnp.pad(terrain_points, ((0, npad - n), (0, 0)),
                constant_values=1e6)
    tT = t.T
    k2 = jnp.sum(t * t, axis=1)[None, :]

    nsteps = npad // TBLK
    cost = pl.pallas_call(
        _body,
        grid=(nsteps,),
        in_specs=[
            pl.BlockSpec((NQ, 3), lambda i: (0, 0)),
            pl.BlockSpec((3, TBLK), lambda i: (0, i)),
            pl.BlockSpec((1, TBLK), lambda i: (0, i)),
        ],
        out_specs=pl.BlockSpec((NQ, 1), lambda i: (0, 0)),
        out_shape=jax.ShapeDtypeStruct((NQ, 1), jnp.float32),
        scratch_shapes=[
            pltpu.VMEM((NQ, 1), jnp.float32),
            pltpu.VMEM((NQ, 1), jnp.float32),
        ],
    )(q, tT, k2)

    return cost.reshape(B, P, T).sum(axis=-1)


# speedup vs baseline: 1.8813x; 1.8715x over previous
import jax
import jax.numpy as jnp
from jax.experimental import pallas as pl
from jax.experimental.pallas import tpu as pltpu

RQ = 2.0
THRESHOLD = 4.0
R2 = (2.0 * RQ) ** 2

TBLK = 2048
NQ = 720


def _body(q_ref, tT_ref, cost_ref, dsum_ref, cnt_ref):
    i = pl.program_id(0)
    nsteps = pl.num_programs(0)

    @pl.when(i == 0)
    def _init():
        dsum_ref[...] = jnp.zeros_like(dsum_ref)
        cnt_ref[...] = jnp.zeros_like(cnt_ref)

    g = jax.lax.dot_general(
        q_ref[...], tT_ref[...], (((1,), (0,)), ((), ())),
        preferred_element_type=jnp.float32)
    x = jnp.maximum(g, 1e-12)
    dist = x * jax.lax.rsqrt(x)
    m = x <= R2
    dist_m = jnp.where(m, dist, 0.0)
    m_f = jnp.where(m, 1.0, 0.0)
    def lane_tree_sum(a):
        cols = [a[:, k * 128:(k + 1) * 128] for k in range(TBLK // 128)]
        while len(cols) > 1:
            cols = [cols[j] + cols[j + 1] for j in range(0, len(cols), 2)]
        return cols[0]

    dsum_ref[...] += lane_tree_sum(dist_m)
    cnt_ref[...] += lane_tree_sum(m_f)

    @pl.when(i == nsteps - 1)
    def _fini():
        cnt = cnt_ref[...].sum(axis=1, keepdims=True)
        dsum = dsum_ref[...].sum(axis=1, keepdims=True)
        d_mean = dsum / jnp.maximum(cnt, 1.0)
        cost = -(d_mean * d_mean) * (1.0 / (RQ * RQ)) + THRESHOLD
        cost_ref[...] = jnp.where(cnt > 0.0, cost, 0.0)


def kernel(predicted_trajectories_global, terrain_points):
    traj = predicted_trajectories_global
    B, P, T, D = traj.shape
    qpts = traj.reshape(-1, D)
    ones = jnp.ones((NQ, 1), jnp.float32)
    q2 = jnp.sum(qpts * qpts, axis=1, keepdims=True) + 1e-12
    q = jnp.concatenate([q2, -2.0 * qpts, ones], axis=1)

    n = terrain_points.shape[0]
    npad = ((n + TBLK - 1) // TBLK) * TBLK
    t = jnp.pad(terrain_points, ((0, npad - n), (0, 0)),
                constant_values=1e6)
    tT = jnp.concatenate(
        [jnp.ones((1, npad), jnp.float32), t.T,
         jnp.sum(t * t, axis=1)[None, :]], axis=0)

    nsteps = npad // TBLK
    cost = pl.pallas_call(
        _body,
        grid=(nsteps,),
        in_specs=[
            pl.BlockSpec((NQ, 5), lambda i: (0, 0)),
            pl.BlockSpec((5, TBLK), lambda i: (0, i)),
        ],
        out_specs=pl.BlockSpec((NQ, 1), lambda i: (0, 0)),
        out_shape=jax.ShapeDtypeStruct((NQ, 1), jnp.float32),
        scratch_shapes=[
            pltpu.VMEM((NQ, 128), jnp.float32),
            pltpu.VMEM((NQ, 128), jnp.float32),
        ],
    )(q, tT)

    return cost.reshape(B, P, T).sum(axis=-1)
